# Initial kernel scaffold; baseline (speedup 1.0000x reference)
import jax
import jax.numpy as jnp
from jax.experimental import pallas as pl


def kernel(x, batch, Ws1, bs1, Wh1, bh1, Wa1, Wb1, bb1, Ws2, bs2, Wh2, bh2, Wa2, Wb2, bb2, Ws3, bs3, Wh3, bh3, Wa3, Wb3, bb3, Ws4, bs4, Wh4, bh4, Wa4, Wb4, bb4, Wout, bout):
    raise NotImplementedError("write your pallas kernel here")


# speedup vs baseline: 6.2229x; 6.2229x over previous
import functools

import jax
import jax.numpy as jnp
from jax.experimental import pallas as pl
from jax.experimental.pallas import tpu as pltpu

_K = 40
_NEG_BIG = -3.0e38


def _row_block(n, target=200):
    best = 8
    for r in range(8, target + 1, 8):
        if n % r == 0:
            best = r
    return best


def _sortable(bits):
    return jnp.where(bits >= 0, bits, jnp.bitwise_xor(bits, jnp.int32(0x7FFFFFFF)))


def _prep_body(x_ref, w_ref, b_ref, sh_ref, sq_ref):
    sh = jnp.dot(x_ref[...], w_ref[...], preferred_element_type=jnp.float32)
    sh = sh + b_ref[...]
    sh_ref[...] = sh
    s = sh[:, :4]
    sq_ref[...] = jnp.sum(s * s, axis=1, keepdims=True)


def _prep(x, Ws, bs, Wh, bh):
    n, cin = x.shape
    rp = _row_block(n, 2000)
    wsh = jnp.concatenate([Ws, Wh], axis=0).T
    bsh = jnp.concatenate([bs, bh])[None, :]
    sh, sq = pl.pallas_call(
        _prep_body,
        grid=(n // rp,),
        in_specs=[
            pl.BlockSpec((rp, cin), lambda i: (i, 0)),
            pl.BlockSpec((cin, 26), lambda i: (0, 0)),
            pl.BlockSpec((1, 26), lambda i: (0, 0)),
        ],
        out_specs=[
            pl.BlockSpec((rp, 26), lambda i: (i, 0)),
            pl.BlockSpec((rp, 1), lambda i: (i, 0)),
        ],
        out_shape=[
            jax.ShapeDtypeStruct((n, 26), jnp.float32),
            jax.ShapeDtypeStruct((n, 1), jnp.float32),
        ],
    )(x, wsh, bsh)
    return sh[:, :4], sh[:, 4:], sq


def _main_body(nch, s_blk_ref, sT_ref, sqc_ref, sqr_ref, h_ref, hT_ref,
               x_ref, waT_ref, wbT_ref, bb_ref, out_ref,
               d2_ref, key_ref, m01_ref):
    n = sT_ref.shape[1]
    d2 = (sqc_ref[...] + sqr_ref[...]
          - 2.0 * jnp.dot(s_blk_ref[...], sT_ref[...],
                          preferred_element_type=jnp.float32))
    key = _sortable(jax.lax.bitcast_convert_type(d2, jnp.int32))
    key_ref[...] = key
    d2_ref[...] = jnp.exp(-10.0 * jnp.maximum(d2, 0.0))

    lo0 = jnp.min(key, axis=1, keepdims=True) - 1
    hi0 = jnp.max(key, axis=1, keepdims=True)

    def step(_, carry):
        lo, hi = carry
        mid = lo + ((hi - lo) >> 1)
        cnt = jnp.sum((key_ref[...] <= mid).astype(jnp.int32), axis=1,
                      keepdims=True)
        ge = cnt >= _K
        return jnp.where(ge, lo, mid), jnp.where(ge, mid, hi)

    _, t = jax.lax.fori_loop(0, 32, step, (lo0, hi0))

    key = key_ref[...]
    selle = key <= t
    seltie = key == t
    m01_ref[...] = selle.astype(jnp.float32)
    cnt_le = jnp.sum(selle.astype(jnp.float32), axis=1, keepdims=True)
    cnt_tie = jnp.sum(seltie.astype(jnp.float32), axis=1, keepdims=True)
    extra = cnt_le - _K

    tval = jax.lax.bitcast_convert_type(_sortable(t), jnp.float32)
    w_t = jnp.exp(-10.0 * jnp.maximum(tval, 0.0))

    w_raw = d2_ref[...]
    wm = w_raw * m01_ref[...]
    sum_wh = jnp.dot(wm, h_ref[...], preferred_element_type=jnp.float32)
    tie_h = jnp.dot(seltie.astype(jnp.float32), h_ref[...],
                    preferred_element_type=jnp.float32)
    mean_agg = (sum_wh - (extra / cnt_tie) * w_t * tie_h) * (1.0 / _K)

    acc = jnp.dot(x_ref[...], waT_ref[...], preferred_element_type=jnp.float32)
    acc = acc + jnp.dot(mean_agg, wbT_ref[:22, :],
                        preferred_element_type=jnp.float32)

    def ch_step(ch, a):
        hrow = hT_ref[pl.ds(ch, 1), :]
        tmp = jnp.where(m01_ref[...] > 0.5, d2_ref[...] * hrow, _NEG_BIG)
        mv = jnp.max(tmp, axis=1, keepdims=True)
        return a + mv * wbT_ref[pl.ds(22 + ch, 1), :]

    acc = jax.lax.fori_loop(0, nch, ch_step, acc)
    out_ref[...] = acc + bb_ref[...]


def _gravnet_layer_pallas(x, Ws, bs, Wh, bh, Wa, Wb, bb):
    n, cin = x.shape
    cout = Wa.shape[0]
    s, h, sq = _prep(x, Ws, bs, Wh, bh)
    sT = s.T
    hT = h.T
    sqr = sq.reshape(1, n)
    waT = Wa.T
    wbT = Wb.T
    bb2 = bb[None, :]
    r = _row_block(n, 100)
    nb = n // r
    out = pl.pallas_call(
        functools.partial(_main_body, 22),
        grid=(nb,),
        in_specs=[
            pl.BlockSpec((r, 4), lambda i: (i, 0)),
            pl.BlockSpec((4, n), lambda i: (0, 0)),
            pl.BlockSpec((r, 1), lambda i: (i, 0)),
            pl.BlockSpec((1, n), lambda i: (0, 0)),
            pl.BlockSpec((n, 22), lambda i: (0, 0)),
            pl.BlockSpec((22, n), lambda i: (0, 0)),
            pl.BlockSpec((r, cin), lambda i: (i, 0)),
            pl.BlockSpec((cin, cout), lambda i: (0, 0)),
            pl.BlockSpec((44, cout), lambda i: (0, 0)),
            pl.BlockSpec((1, cout), lambda i: (0, 0)),
        ],
        out_specs=pl.BlockSpec((r, cout), lambda i: (i, 0)),
        out_shape=jax.ShapeDtypeStruct((n, cout), jnp.float32),
        scratch_shapes=[
            pltpu.VMEM((r, n), jnp.float32),
            pltpu.VMEM((r, n), jnp.int32),
            pltpu.VMEM((r, n), jnp.float32),
        ],
    )(s, sT, sq, sqr, h, hT, x, waT, wbT, bb2)
    return out


def _final_body(x1_ref, x2_ref, x3_ref, x4_ref, w1_ref, w2_ref, w3_ref,
                w4_ref, b_ref, out_ref):
    acc = jnp.dot(x1_ref[...], w1_ref[...], preferred_element_type=jnp.float32)
    acc += jnp.dot(x2_ref[...], w2_ref[...], preferred_element_type=jnp.float32)
    acc += jnp.dot(x3_ref[...], w3_ref[...], preferred_element_type=jnp.float32)
    acc += jnp.dot(x4_ref[...], w4_ref[...], preferred_element_type=jnp.float32)
    out_ref[...] = jnp.maximum(acc + b_ref[...], 0.0)


def _final(x1, x2, x3, x4, Wout, bout):
    n = x1.shape[0]
    cf = Wout.shape[0]
    d1, d2_, d3, d4 = x1.shape[1], x2.shape[1], x3.shape[1], x4.shape[1]
    w = Wout.T
    w1, w2, w3, w4 = (w[:d1], w[d1:d1 + d2_], w[d1 + d2_:d1 + d2_ + d3],
                      w[d1 + d2_ + d3:])
    rp = _row_block(n, 2000)
    return pl.pallas_call(
        _final_body,
        grid=(n // rp,),
        in_specs=[
            pl.BlockSpec((rp, d1), lambda i: (i, 0)),
            pl.BlockSpec((rp, d2_), lambda i: (i, 0)),
            pl.BlockSpec((rp, d3), lambda i: (i, 0)),
            pl.BlockSpec((rp, d4), lambda i: (i, 0)),
            pl.BlockSpec((d1, cf), lambda i: (0, 0)),
            pl.BlockSpec((d2_, cf), lambda i: (0, 0)),
            pl.BlockSpec((d3, cf), lambda i: (0, 0)),
            pl.BlockSpec((d4, cf), lambda i: (0, 0)),
            pl.BlockSpec((1, cf), lambda i: (0, 0)),
        ],
        out_specs=pl.BlockSpec((rp, cf), lambda i: (i, 0)),
        out_shape=jax.ShapeDtypeStruct((n, cf), jnp.float32),
    )(x1, x2, x3, x4, w1, w2, w3, w4, bout[None, :])


def kernel(x, batch,
           Ws1, bs1, Wh1, bh1, Wa1, Wb1, bb1,
           Ws2, bs2, Wh2, bh2, Wa2, Wb2, bb2,
           Ws3, bs3, Wh3, bh3, Wa3, Wb3, bb3,
           Ws4, bs4, Wh4, bh4, Wa4, Wb4, bb4,
           Wout, bout):
    x1 = _gravnet_layer_pallas(x, Ws1, bs1, Wh1, bh1, Wa1, Wb1, bb1)
    x2 = _gravnet_layer_pallas(x1, Ws2, bs2, Wh2, bh2, Wa2, Wb2, bb2)
    x3 = _gravnet_layer_pallas(x2, Ws3, bs3, Wh3, bh3, Wa3, Wb3, bb3)
    x4 = _gravnet_layer_pallas(x3, Ws4, bs4, Wh4, bh4, Wa4, Wb4, bb4)
    return _final(x1, x2, x3, x4, Wout, bout)
